# baseline (speedup 1.0000x reference)
import functools

import jax
import jax.numpy as jnp
from jax import lax
from jax.experimental import pallas as pl
from jax.experimental.pallas import tpu as pltpu
from jax.experimental.pallas import tpu_sc as plsc

_NUM_TOKEN_ID = 5
_NC, _NS, _L = 2, 16, 16
_NW = _NC * _NS
_R = 24
_LN2 = 0.6931471805599453


def _log1p16(x):
  y = x + 1.0
  bits = lax.bitcast_convert_type(y, jnp.int32)
  ex = (lax.shift_right_logical(bits, 23) - 127).astype(jnp.float32)
  m = lax.bitcast_convert_type(
      (bits & jnp.int32(0x007FFFFF)) | jnp.int32(0x3F800000), jnp.float32)
  t = (m - 1.0) / (m + 1.0)
  t2 = t * t
  s = 1.0 / 9.0 + t2 * 0.0
  s = s * t2 + 1.0 / 7.0
  s = s * t2 + 1.0 / 5.0
  s = s * t2 + 1.0 / 3.0
  s = s * t2 + 1.0
  return ex * _LN2 + 2.0 * t * s


def _sc_kernel(N, H, S_dim):
  tok_w = N // _NW
  assert S_dim % tok_w == 0
  nbig = tok_w // _R
  tail = tok_w - nbig * _R
  mesh = plsc.VectorSubcoreMesh(
      core_axis_name="c", subcore_axis_name="s",
      num_cores=_NC, num_subcores=_NS)

  @functools.partial(
      pl.kernel,
      out_type=jax.ShapeDtypeStruct((N, H), jnp.float32),
      mesh=mesh,
      scratch_types=[
          pltpu.VMEM((tok_w + _L,), jnp.int32),
          pltpu.VMEM((tok_w + _L,), jnp.float32),
          pltpu.VMEM((H,), jnp.float32),
          pltpu.VMEM((_R, H), jnp.float32),
          pltpu.VMEM((_R, H), jnp.float32),
          pltpu.SemaphoreType.DMA,
          pltpu.SemaphoreType.DMA,
          pltpu.SemaphoreType.DMA,
          pltpu.SemaphoreType.DMA,
          pltpu.SemaphoreType.DMA,
      ],
  )
  def body(ids_hbm, vals_hbm, table_hbm, dir_hbm, out_hbm,
           idx_v, vals_v, dir_v, buf0, buf1, gsem0, gsem1, ssem0, ssem1, psem):
    bufs = (buf0, buf1)
    gsems = (gsem0, gsem1)
    ssems = (ssem0, ssem1)
    wid = lax.axis_index("s") * _NC + lax.axis_index("c")
    base = pl.multiple_of(wid * tok_w, 8)
    b_row = base // S_dim
    s_col = pl.multiple_of(base % S_dim, 8)
    pltpu.sync_copy(ids_hbm.at[b_row, pl.ds(s_col, tok_w)],
                    idx_v.at[pl.ds(0, tok_w)])
    vcp = pltpu.async_copy(vals_hbm.at[b_row, pl.ds(s_col, tok_w)],
                           vals_v.at[pl.ds(0, tok_w)], psem)
    dcp = pltpu.async_copy(dir_hbm, dir_v, psem)

    def start_gather(off, rows, p):
      pltpu.async_copy(table_hbm.at[idx_v.at[pl.ds(off, rows)]],
                       bufs[p].at[pl.ds(0, rows)], gsems[p])

    def wait_gather(rows, p):
      pltpu.make_async_copy(table_hbm.at[idx_v.at[pl.ds(0, rows)]],
                            bufs[p].at[pl.ds(0, rows)], gsems[p]).wait()

    def start_scatter(off, rows, p):
      pltpu.async_copy(bufs[p].at[pl.ds(0, rows)],
                       out_hbm.at[pl.ds(base + off, rows)], ssems[p])

    def wait_scatter(rows, p):
      pltpu.make_async_copy(bufs[p].at[pl.ds(0, rows)],
                            out_hbm.at[pl.ds(base, rows)], ssems[p]).wait()

    def fma_rows(off, rows, p):
      buf_ref = bufs[p]

      def row_body(r, _):
        id_r = idx_v[pl.ds(off + r, _L)][0]

        @pl.when(id_r == _NUM_TOKEN_ID)
        def _():
          v16 = vals_v[pl.ds(off + r, _L)]
          c16 = jnp.sign(v16) * _log1p16(jnp.abs(v16))
          c_r = c16[0]
          cb = jnp.full((_L,), c_r, jnp.float32)

          def col_body(jj, _):
            sl = pl.ds(jj * _L, _L)
            buf_ref[r, sl] = buf_ref[r, sl] + cb * dir_v[sl]
            return ()
          lax.fori_loop(0, H // _L, col_body, ())
        return ()

      lax.fori_loop(0, rows, row_body, ())

    start_gather(0, _R, 0)
    start_gather(_R, _R, 1)
    vcp.wait()
    dcp.wait()

    npair = nbig // 2

    def pair_body(k, _):
      off0 = pl.multiple_of(k * (2 * _R), 8)
      off1 = off0 + _R
      wait_gather(_R, 0)
      fma_rows(off0, _R, 0)
      start_scatter(off0, _R, 0)
      wait_gather(_R, 1)
      fma_rows(off1, _R, 1)
      start_scatter(off1, _R, 1)
      wait_scatter(_R, 0)
      start_gather(off0 + 2 * _R, _R, 0)

      @pl.when(k + 1 < npair)
      def _():
        wait_scatter(_R, 1)
        start_gather(off1 + 2 * _R, _R, 1)
      return ()

    lax.fori_loop(0, npair, pair_body, ())

    off20 = (nbig - 1) * _R
    wait_gather(_R, 0)
    fma_rows(off20, _R, 0)
    start_scatter(off20, _R, 0)
    wait_scatter(_R, 1)
    start_gather(nbig * _R, tail, 1)
    wait_gather(tail, 1)
    fma_rows(nbig * _R, tail, 1)
    start_scatter(nbig * _R, tail, 1)
    wait_scatter(_R, 0)
    wait_scatter(tail, 1)

  return body


def kernel(input_ids, numerical_values, embedding_table, numerical_direction):
  B, S = input_ids.shape
  V, H = embedding_table.shape
  N = B * S
  ids = input_ids.astype(jnp.int32)
  vals = numerical_values.astype(jnp.float32)
  out = _sc_kernel(N, H, S)(ids, vals, embedding_table, numerical_direction)
  return out.reshape(B, S, H)


# speedup vs baseline: 1.0328x; 1.0328x over previous
import functools

import jax
import jax.numpy as jnp
from jax import lax
from jax.experimental import pallas as pl
from jax.experimental.pallas import tpu as pltpu
from jax.experimental.pallas import tpu_sc as plsc

_NUM_TOKEN_ID = 5
_NC, _NS, _L = 2, 16, 16
_NW = _NC * _NS
_R = 24
_LN2 = 0.6931471805599453


def _log1p16(x):
  y = x + 1.0
  bits = lax.bitcast_convert_type(y, jnp.int32)
  ex = (lax.shift_right_logical(bits, 23) - 127).astype(jnp.float32)
  m = lax.bitcast_convert_type(
      (bits & jnp.int32(0x007FFFFF)) | jnp.int32(0x3F800000), jnp.float32)
  t = (m - 1.0) / (m + 1.0)
  t2 = t * t
  s = 1.0 / 9.0 + t2 * 0.0
  s = s * t2 + 1.0 / 7.0
  s = s * t2 + 1.0 / 5.0
  s = s * t2 + 1.0 / 3.0
  s = s * t2 + 1.0
  return ex * _LN2 + 2.0 * t * s


def _sc_kernel(N, H, S_dim):
  tok_w = N // _NW
  assert S_dim % tok_w == 0
  nbig = tok_w // _R
  tail = tok_w - nbig * _R
  mesh = plsc.VectorSubcoreMesh(
      core_axis_name="c", subcore_axis_name="s",
      num_cores=_NC, num_subcores=_NS)

  @functools.partial(
      pl.kernel,
      out_type=jax.ShapeDtypeStruct((N, H), jnp.float32),
      mesh=mesh,
      scratch_types=[
          pltpu.VMEM((tok_w + _L,), jnp.int32),
          pltpu.VMEM((tok_w + _L,), jnp.float32),
          pltpu.VMEM((H,), jnp.float32),
          pltpu.VMEM((_R, H), jnp.float32),
          pltpu.VMEM((_R, H), jnp.float32),
          pltpu.SemaphoreType.DMA,
          pltpu.SemaphoreType.DMA,
          pltpu.SemaphoreType.DMA,
          pltpu.SemaphoreType.DMA,
          pltpu.SemaphoreType.DMA,
      ],
  )
  def body(ids_hbm, vals_hbm, table_hbm, dir_hbm, out_hbm,
           idx_v, vals_v, dir_v, buf0, buf1, gsem0, gsem1, ssem0, ssem1, psem):
    bufs = (buf0, buf1)
    gsems = (gsem0, gsem1)
    ssems = (ssem0, ssem1)
    wid = lax.axis_index("s") * _NC + lax.axis_index("c")
    base = pl.multiple_of(wid * tok_w, 8)
    b_row = base // S_dim
    s_col = pl.multiple_of(base % S_dim, 8)
    pltpu.sync_copy(ids_hbm.at[b_row, pl.ds(s_col, tok_w)],
                    idx_v.at[pl.ds(0, tok_w)])
    vcp = pltpu.async_copy(vals_hbm.at[b_row, pl.ds(s_col, tok_w)],
                           vals_v.at[pl.ds(0, tok_w)], psem)
    dcp = pltpu.async_copy(dir_hbm, dir_v, psem)

    def start_gather(off, rows, p):
      pltpu.async_copy(table_hbm.at[idx_v.at[pl.ds(off, rows)]],
                       bufs[p].at[pl.ds(0, rows)], gsems[p])

    def wait_gather(rows, p):
      pltpu.make_async_copy(table_hbm.at[idx_v.at[pl.ds(0, rows)]],
                            bufs[p].at[pl.ds(0, rows)], gsems[p]).wait()

    def start_scatter(off, rows, p):
      pltpu.async_copy(bufs[p].at[pl.ds(0, rows)],
                       out_hbm.at[pl.ds(base + off, rows)], ssems[p])

    def wait_scatter(rows, p):
      pltpu.make_async_copy(bufs[p].at[pl.ds(0, rows)],
                            out_hbm.at[pl.ds(base, rows)], ssems[p]).wait()

    def fma_rows(off, rows, p):
      buf_ref = bufs[p]

      def row_body(r, _):
        id_r = idx_v[pl.ds(off + r, _L)][0]

        @pl.when(id_r == _NUM_TOKEN_ID)
        def _():
          v16 = vals_v[pl.ds(off + r, _L)]
          c16 = jnp.sign(v16) * _log1p16(jnp.abs(v16))
          c_r = c16[0]
          cb = jnp.full((_L,), c_r, jnp.float32)

          def col_body(jj, _):
            sl = pl.ds(jj * _L, _L)
            buf_ref[r, sl] = buf_ref[r, sl] + cb * dir_v[sl]
            return ()
          lax.fori_loop(0, H // _L, col_body, ())
        return ()

      lax.fori_loop(0, rows, row_body, ())

    start_gather(0, _R, 0)
    start_gather(_R, _R, 1)
    vcp.wait()
    dcp.wait()

    npair = nbig // 2

    def pair_body(k, _):
      off0 = pl.multiple_of(k * (2 * _R), 8)
      wait_gather(_R, 0)
      fma_rows(off0, _R, 0)
      start_scatter(off0, _R, 0)
      wait_scatter(_R, 0)
      start_gather(off0 + 2 * _R, _R, 0)
      off1 = off0 + _R
      wait_gather(_R, 1)
      fma_rows(off1, _R, 1)
      start_scatter(off1, _R, 1)

      @pl.when(k + 1 < npair)
      def _():
        wait_scatter(_R, 1)
        start_gather(off1 + 2 * _R, _R, 1)
      return ()

    lax.fori_loop(0, npair, pair_body, ())

    off20 = (nbig - 1) * _R
    wait_gather(_R, 0)
    fma_rows(off20, _R, 0)
    start_scatter(off20, _R, 0)
    wait_scatter(_R, 1)
    start_gather(nbig * _R, tail, 1)
    wait_gather(tail, 1)
    fma_rows(nbig * _R, tail, 1)
    start_scatter(nbig * _R, tail, 1)
    wait_scatter(_R, 0)
    wait_scatter(tail, 1)

  return body


def kernel(input_ids, numerical_values, embedding_table, numerical_direction):
  B, S = input_ids.shape
  V, H = embedding_table.shape
  N = B * S
  ids = input_ids.astype(jnp.int32)
  vals = numerical_values.astype(jnp.float32)
  out = _sc_kernel(N, H, S)(ids, vals, embedding_table, numerical_direction)
  return out.reshape(B, S, H)
